# baseline (speedup 1.0000x reference)
import functools

import jax
import jax.numpy as jnp
from jax import lax
from jax.experimental import pallas as pl
from jax.experimental.pallas import tpu as pltpu
from jax.experimental.pallas import tpu_sc as plsc

N = 10000
E = 320000
D = 128
DH = D // 2

NC = 2
NS = 16
NW = NC * NS
N_PAD = 10240
RPT = N_PAD // NS
CH = 128
E_PAD = 327680
NCHD = E_PAD // (NW * CH)
NCHS = E_PAD // (NS * CH)
GROUP = 5
W_IDX = 20
BM = 256
NBLK = N_PAD // BM


def _sc_degree(dst2d):
    mesh = plsc.VectorSubcoreMesh(core_axis_name="c", subcore_axis_name="s")

    @functools.partial(
        pl.kernel,
        out_type=jax.ShapeDtypeStruct((NC, N_PAD), jnp.float32),
        mesh=mesh,
        compiler_params=pltpu.CompilerParams(use_tc_tiling_on_sc=False),
        scratch_types=[
            pltpu.VMEM((NCHD, CH), jnp.int32),
            pltpu.VMEM((CH,), jnp.float32),
            pltpu.VMEM((CH,), jnp.float32),
            pltpu.VMEM_SHARED((N_PAD,), jnp.float32),
        ],
    )
    def deg_kernel(dst_hbm, out_hbm, idx_v, ones_v, zero_v, acc):
        c = lax.axis_index("c")
        s = lax.axis_index("s")
        w = s * NC + c
        for t in range(CH // 16):
            ones_v[pl.ds(t * 16, 16)] = jnp.full((16,), 1.0, jnp.float32)
            zero_v[pl.ds(t * 16, 16)] = jnp.zeros((16,), jnp.float32)
        pltpu.sync_copy(dst_hbm.at[pl.ds(w * NCHD, NCHD)], idx_v)
        for q in range(RPT // CH):
            pltpu.sync_copy(zero_v, acc.at[pl.ds(s * RPT + q * CH, CH)])
        plsc.subcore_barrier()

        def body(j, carry):
            pltpu.sync_copy(ones_v, acc.at[idx_v.at[j]], add=True)
            return carry

        lax.fori_loop(0, NCHD, body, 0)
        plsc.subcore_barrier()
        pltpu.sync_copy(acc.at[pl.ds(s * RPT, RPT)],
                        out_hbm.at[c, pl.ds(s * RPT, RPT)])

    return deg_kernel(dst2d)


def _sc_scatter(xlo, xhi, src2d, dst2d, zrows):
    mesh = plsc.VectorSubcoreMesh(core_axis_name="c", subcore_axis_name="s")

    @functools.partial(
        pl.kernel,
        out_type=jax.ShapeDtypeStruct((NC, N_PAD, DH), jnp.float32),
        mesh=mesh,
        compiler_params=pltpu.CompilerParams(use_tc_tiling_on_sc=False),
        scratch_types=(
            [pltpu.VMEM((W_IDX, CH), jnp.int32),
             pltpu.VMEM((W_IDX, CH), jnp.int32)]
            + [pltpu.VMEM((CH, DH), jnp.float32)] * GROUP
            + [pltpu.VMEM_SHARED((N_PAD, DH), jnp.float32),
               pltpu.VMEM_SHARED((N_PAD, DH), jnp.float32)]
            + [pltpu.SemaphoreType.DMA] * (2 * GROUP)
        ),
    )
    def scat_kernel(xlo_hbm, xhi_hbm, src_hbm, dst_hbm, z_hbm, out_hbm,
                    src_w, dst_w, *rest):
        bufs = rest[:GROUP]
        acc = rest[GROUP]
        xsp = rest[GROUP + 1]
        gsem = rest[GROUP + 2:2 * GROUP + 2]
        tsem = rest[2 * GROUP + 2:]
        c = lax.axis_index("c")
        s = lax.axis_index("s")
        pltpu.sync_copy(z_hbm, acc.at[pl.ds(s * RPT, RPT)])

        @pl.when(c == 0)
        def _():
            pltpu.sync_copy(xlo_hbm.at[pl.ds(s * RPT, RPT)],
                            xsp.at[pl.ds(s * RPT, RPT)])

        @pl.when(c == 1)
        def _():
            pltpu.sync_copy(xhi_hbm.at[pl.ds(s * RPT, RPT)],
                            xsp.at[pl.ds(s * RPT, RPT)])

        plsc.subcore_barrier()

        def wbody(wi, carry):
            pltpu.sync_copy(src_hbm.at[pl.ds(s * NCHS + wi * W_IDX, W_IDX)],
                            src_w)
            pltpu.sync_copy(dst_hbm.at[pl.ds(s * NCHS + wi * W_IDX, W_IDX)],
                            dst_w)

            def body(g, carry2):
                j = g * GROUP
                ds = []
                for k in range(GROUP):
                    @pl.when(g > 0)
                    def _(k=k):
                        pltpu.make_async_copy(
                            bufs[k], acc.at[dst_w.at[j - GROUP + k]],
                            tsem[k]).wait()

                    ds.append(pltpu.async_copy(xsp.at[src_w.at[j + k]],
                                               bufs[k], gsem[k]))
                for k in range(GROUP):
                    ds[k].wait()
                    pltpu.async_copy(
                        bufs[k], acc.at[dst_w.at[j + k]], tsem[k], add=True)
                return carry2

            lax.fori_loop(0, W_IDX // GROUP, body, 0)
            for k in range(GROUP):
                pltpu.make_async_copy(
                    bufs[k], acc.at[dst_w.at[W_IDX - GROUP + k]],
                    tsem[k]).wait()
            return carry

        lax.fori_loop(0, NCHS // W_IDX, wbody, 0)
        plsc.subcore_barrier()
        pltpu.sync_copy(acc.at[pl.ds(s * RPT, RPT)],
                        out_hbm.at[c, pl.ds(s * RPT, RPT)])

    return scat_kernel(xlo, xhi, src2d, dst2d, zrows)


def _dis_from_parts(dp_block):
    return lax.rsqrt(dp_block[0] + dp_block[1] + 1.0)


def _mat_scale(x, W, deg_parts):
    def body(x_ref, w_ref, dp_ref, lo_ref, hi_ref):
        dis = _dis_from_parts(dp_ref[...])
        y = jnp.dot(x_ref[...], w_ref[...], preferred_element_type=jnp.float32)
        y = y * dis[:, None]
        lo_ref[...] = y[:, :DH]
        hi_ref[...] = y[:, DH:]

    return pl.pallas_call(
        body,
        grid=(NBLK,),
        in_specs=[
            pl.BlockSpec((BM, D), lambda i: (i, 0)),
            pl.BlockSpec((D, D), lambda i: (0, 0)),
            pl.BlockSpec((NC, BM), lambda i: (0, i)),
        ],
        out_specs=[
            pl.BlockSpec((BM, DH), lambda i: (i, 0)),
            pl.BlockSpec((BM, DH), lambda i: (i, 0)),
        ],
        out_shape=[
            jax.ShapeDtypeStruct((N_PAD, DH), jnp.float32),
            jax.ShapeDtypeStruct((N_PAD, DH), jnp.float32),
        ],
    )(x, W, deg_parts)


def _bn_stats(accp, xlo, xhi, deg_parts, b):
    def body(a_ref, lo_ref, hi_ref, dp_ref, b_ref, h_ref, s_ref):
        i = pl.program_id(0)
        dis = _dis_from_parts(dp_ref[...])
        agg = jnp.concatenate([a_ref[0] + lo_ref[...],
                               a_ref[1] + hi_ref[...]], axis=-1)
        t = agg * dis[:, None] + b_ref[...]
        h = jnp.maximum(t, 0.0)
        rid = i * BM + lax.broadcasted_iota(jnp.int32, (BM, 1), 0)
        h = jnp.where(rid < N, h, 0.0)
        h_ref[...] = h
        ps = jnp.concatenate(
            [jnp.sum(h, axis=0, keepdims=True),
             jnp.sum(h * h, axis=0, keepdims=True),
             jnp.zeros((6, D), jnp.float32)], axis=0)

        @pl.when(i == 0)
        def _():
            s_ref[...] = ps

        @pl.when(i > 0)
        def _():
            s_ref[...] = s_ref[...] + ps

    return pl.pallas_call(
        body,
        grid=(NBLK,),
        in_specs=[
            pl.BlockSpec((NC, BM, DH), lambda i: (0, i, 0)),
            pl.BlockSpec((BM, DH), lambda i: (i, 0)),
            pl.BlockSpec((BM, DH), lambda i: (i, 0)),
            pl.BlockSpec((NC, BM), lambda i: (0, i)),
            pl.BlockSpec((1, D), lambda i: (0, 0)),
        ],
        out_specs=[
            pl.BlockSpec((BM, D), lambda i: (i, 0)),
            pl.BlockSpec((8, D), lambda i: (0, 0)),
        ],
        out_shape=[
            jax.ShapeDtypeStruct((N_PAD, D), jnp.float32),
            jax.ShapeDtypeStruct((8, D), jnp.float32),
        ],
    )(accp, xlo, xhi, deg_parts, b.reshape(1, D))


def _bn_norm(h_blk, s_blk, g_blk, be_blk, rid):
    mu = s_blk[0] * (1.0 / N)
    var = s_blk[1] * (1.0 / N) - mu * mu
    inv = 1.0 / jnp.sqrt(var + 1e-5)
    xn = g_blk * (h_blk - mu) * inv + be_blk
    return jnp.where(rid < N, xn, 0.0)


def _bn_apply_mat(h, stats, gamma, beta, Wn, deg_parts):
    def body(h_ref, s_ref, g_ref, be_ref, w_ref, dp_ref,
             xn_ref, lo_ref, hi_ref):
        i = pl.program_id(0)
        rid = i * BM + lax.broadcasted_iota(jnp.int32, (BM, 1), 0)
        xn = _bn_norm(h_ref[...], s_ref[...], g_ref[...], be_ref[...], rid)
        xn_ref[...] = xn
        dis = _dis_from_parts(dp_ref[...])
        y = jnp.dot(xn, w_ref[...], preferred_element_type=jnp.float32)
        y = y * dis[:, None]
        lo_ref[...] = y[:, :DH]
        hi_ref[...] = y[:, DH:]

    return pl.pallas_call(
        body,
        grid=(NBLK,),
        in_specs=[
            pl.BlockSpec((BM, D), lambda i: (i, 0)),
            pl.BlockSpec((8, D), lambda i: (0, 0)),
            pl.BlockSpec((1, D), lambda i: (0, 0)),
            pl.BlockSpec((1, D), lambda i: (0, 0)),
            pl.BlockSpec((D, D), lambda i: (0, 0)),
            pl.BlockSpec((NC, BM), lambda i: (0, i)),
        ],
        out_specs=[
            pl.BlockSpec((BM, D), lambda i: (i, 0)),
            pl.BlockSpec((BM, DH), lambda i: (i, 0)),
            pl.BlockSpec((BM, DH), lambda i: (i, 0)),
        ],
        out_shape=[
            jax.ShapeDtypeStruct((N_PAD, D), jnp.float32),
            jax.ShapeDtypeStruct((N_PAD, DH), jnp.float32),
            jax.ShapeDtypeStruct((N_PAD, DH), jnp.float32),
        ],
    )(h, stats, gamma.reshape(1, D), beta.reshape(1, D), Wn, deg_parts)


def _head(h3, st3, gamma3, beta3, x1, x2, lin_W, lin_b):
    def body(h_ref, s_ref, g_ref, be_ref, x1_ref, x2_ref, w_ref, lb_ref, o_ref):
        i = pl.program_id(0)
        rid = i * BM + lax.broadcasted_iota(jnp.int32, (BM, 1), 0)
        x3 = _bn_norm(h_ref[...], s_ref[...], g_ref[...], be_ref[...], rid)
        xc = jnp.concatenate([x1_ref[...], x2_ref[...], x3], axis=-1)
        y = jnp.dot(xc, w_ref[...], preferred_element_type=jnp.float32)
        o_ref[...] = jnp.maximum(y + lb_ref[...], 0.0)

    return pl.pallas_call(
        body,
        grid=(NBLK,),
        in_specs=[
            pl.BlockSpec((BM, D), lambda i: (i, 0)),
            pl.BlockSpec((8, D), lambda i: (0, 0)),
            pl.BlockSpec((1, D), lambda i: (0, 0)),
            pl.BlockSpec((1, D), lambda i: (0, 0)),
            pl.BlockSpec((BM, D), lambda i: (i, 0)),
            pl.BlockSpec((BM, D), lambda i: (i, 0)),
            pl.BlockSpec((3 * D, D), lambda i: (0, 0)),
            pl.BlockSpec((1, D), lambda i: (0, 0)),
        ],
        out_specs=pl.BlockSpec((BM, D), lambda i: (i, 0)),
        out_shape=jax.ShapeDtypeStruct((N_PAD, D), jnp.float32),
    )(h3, st3, gamma3.reshape(1, D), beta3.reshape(1, D), x1, x2,
      lin_W, lin_b.reshape(1, D))


def kernel(x, edge_index, batch, W1, b1, gamma1, beta1, W2, b2, gamma2,
           beta2, W3, b3, gamma3, beta3, lin_W, lin_b):
    x_p = jnp.pad(x, ((0, N_PAD - N), (0, 0)))
    padv = jnp.full((E_PAD - E,), N, jnp.int32)
    src2d = jnp.concatenate([edge_index[0], padv]).reshape(NS * NCHS, CH)
    dst2d = jnp.concatenate([edge_index[1], padv]).reshape(NS * NCHS, CH)
    zrows = jnp.zeros((RPT, DH), jnp.float32)

    deg_parts = _sc_degree(dst2d)

    lo1, hi1 = _mat_scale(x_p, W1, deg_parts)
    acc1 = _sc_scatter(lo1, hi1, src2d, dst2d, zrows)
    h1, st1 = _bn_stats(acc1, lo1, hi1, deg_parts, b1)

    x1, lo2, hi2 = _bn_apply_mat(h1, st1, gamma1, beta1, W2, deg_parts)
    acc2 = _sc_scatter(lo2, hi2, src2d, dst2d, zrows)
    h2, st2 = _bn_stats(acc2, lo2, hi2, deg_parts, b2)

    x2, lo3, hi3 = _bn_apply_mat(h2, st2, gamma2, beta2, W3, deg_parts)
    acc3 = _sc_scatter(lo3, hi3, src2d, dst2d, zrows)
    h3, st3 = _bn_stats(acc3, lo3, hi3, deg_parts, b3)

    out = _head(h3, st3, gamma3, beta3, x1, x2, lin_W, lin_b)
    return out[:N]


# speedup vs baseline: 1.0252x; 1.0252x over previous
import functools

import jax
import jax.numpy as jnp
from jax import lax
from jax.experimental import pallas as pl
from jax.experimental.pallas import tpu as pltpu
from jax.experimental.pallas import tpu_sc as plsc

N = 10000
E = 320000
D = 128
DH = D // 2

NC = 2
NS = 16
NW = NC * NS
N_PAD = 10240
RPT = N_PAD // NS
CH = 128
E_PAD = 327680
NCHD = E_PAD // (NW * CH)
NCHS = E_PAD // (NS * CH)
GROUP = 4
W_IDX = 20
BM = 256
NBLK = N_PAD // BM


def _sc_degree(dst2d):
    mesh = plsc.VectorSubcoreMesh(core_axis_name="c", subcore_axis_name="s")

    @functools.partial(
        pl.kernel,
        out_type=jax.ShapeDtypeStruct((NC, N_PAD), jnp.float32),
        mesh=mesh,
        compiler_params=pltpu.CompilerParams(use_tc_tiling_on_sc=False),
        scratch_types=[
            pltpu.VMEM((NCHD, CH), jnp.int32),
            pltpu.VMEM((CH,), jnp.float32),
            pltpu.VMEM((CH,), jnp.float32),
            pltpu.VMEM_SHARED((N_PAD,), jnp.float32),
        ],
    )
    def deg_kernel(dst_hbm, out_hbm, idx_v, ones_v, zero_v, acc):
        c = lax.axis_index("c")
        s = lax.axis_index("s")
        w = s * NC + c
        for t in range(CH // 16):
            ones_v[pl.ds(t * 16, 16)] = jnp.full((16,), 1.0, jnp.float32)
            zero_v[pl.ds(t * 16, 16)] = jnp.zeros((16,), jnp.float32)
        pltpu.sync_copy(dst_hbm.at[pl.ds(w * NCHD, NCHD)], idx_v)
        for q in range(RPT // CH):
            pltpu.sync_copy(zero_v, acc.at[pl.ds(s * RPT + q * CH, CH)])
        plsc.subcore_barrier()

        def body(j, carry):
            pltpu.sync_copy(ones_v, acc.at[idx_v.at[j]], add=True)
            return carry

        lax.fori_loop(0, NCHD, body, 0)
        plsc.subcore_barrier()
        pltpu.sync_copy(acc.at[pl.ds(s * RPT, RPT)],
                        out_hbm.at[c, pl.ds(s * RPT, RPT)])

    return deg_kernel(dst2d)


def _sc_scatter(xlo, xhi, e2d, zrows):
    mesh = plsc.VectorSubcoreMesh(core_axis_name="c", subcore_axis_name="s")

    @functools.partial(
        pl.kernel,
        out_type=jax.ShapeDtypeStruct((NC, N_PAD, DH), jnp.float32),
        mesh=mesh,
        compiler_params=pltpu.CompilerParams(use_tc_tiling_on_sc=False),
        scratch_types=(
            [pltpu.VMEM((W_IDX, 2, CH), jnp.int32),
             pltpu.VMEM((W_IDX, 2, CH), jnp.int32)]
            + [pltpu.VMEM((CH, DH), jnp.float32)] * GROUP
            + [pltpu.VMEM_SHARED((N_PAD, DH), jnp.float32),
               pltpu.VMEM_SHARED((N_PAD, DH), jnp.float32)]
            + [pltpu.SemaphoreType.DMA] * (2 * GROUP + 2)
        ),
    )
    def scat_kernel(xlo_hbm, xhi_hbm, e_hbm, z_hbm, out_hbm,
                    slab0, slab1, *rest):
        bufs = rest[:GROUP]
        acc = rest[GROUP]
        xsp = rest[GROUP + 1]
        gsem = rest[GROUP + 2:2 * GROUP + 2]
        tsem = rest[2 * GROUP + 2:3 * GROUP + 2]
        isem = rest[3 * GROUP + 2:]
        c = lax.axis_index("c")
        s = lax.axis_index("s")
        base0 = s * NCHS
        pltpu.sync_copy(z_hbm, acc.at[pl.ds(s * RPT, RPT)])
        pltpu.async_copy(e_hbm.at[pl.ds(base0, W_IDX)], slab0, isem[0])

        @pl.when(c == 0)
        def _():
            pltpu.sync_copy(xlo_hbm.at[pl.ds(s * RPT, RPT)],
                            xsp.at[pl.ds(s * RPT, RPT)])

        @pl.when(c == 1)
        def _():
            pltpu.sync_copy(xhi_hbm.at[pl.ds(s * RPT, RPT)],
                            xsp.at[pl.ds(s * RPT, RPT)])

        plsc.subcore_barrier()

        def ring(slab):
            def body(g, carry2):
                j = g * GROUP
                ds = []
                for k in range(GROUP):
                    @pl.when(g > 0)
                    def _(k=k):
                        pltpu.make_async_copy(
                            bufs[k], acc.at[slab.at[j - GROUP + k, 1]],
                            tsem[k]).wait()

                    ds.append(pltpu.async_copy(xsp.at[slab.at[j + k, 0]],
                                               bufs[k], gsem[k]))
                for k in range(GROUP):
                    ds[k].wait()
                    pltpu.async_copy(
                        bufs[k], acc.at[slab.at[j + k, 1]], tsem[k], add=True)
                return carry2

            lax.fori_loop(0, W_IDX // GROUP, body, 0)
            for k in range(GROUP):
                pltpu.make_async_copy(
                    bufs[k], acc.at[slab.at[W_IDX - GROUP + k, 1]],
                    tsem[k]).wait()

        npair = NCHS // (2 * W_IDX)

        def wpair(m, carry):
            base = base0 + m * 2 * W_IDX
            pltpu.make_async_copy(e_hbm.at[pl.ds(base, W_IDX)],
                                  slab0, isem[0]).wait()
            pltpu.async_copy(e_hbm.at[pl.ds(base + W_IDX, W_IDX)],
                             slab1, isem[1])
            ring(slab0)
            pltpu.make_async_copy(e_hbm.at[pl.ds(base + W_IDX, W_IDX)],
                                  slab1, isem[1]).wait()

            @pl.when(m + 1 < npair)
            def _():
                pltpu.async_copy(e_hbm.at[pl.ds(base + 2 * W_IDX, W_IDX)],
                                 slab0, isem[0])

            ring(slab1)
            return carry

        lax.fori_loop(0, npair, wpair, 0)
        plsc.subcore_barrier()
        pltpu.sync_copy(acc.at[pl.ds(s * RPT, RPT)],
                        out_hbm.at[c, pl.ds(s * RPT, RPT)])

    return scat_kernel(xlo, xhi, e2d, zrows)


def _dis_from_parts(dp_block):
    return lax.rsqrt(dp_block[0] + dp_block[1] + 1.0)


def _mat_scale(x, W, deg_parts):
    def body(x_ref, w_ref, dp_ref, lo_ref, hi_ref):
        dis = _dis_from_parts(dp_ref[...])
        y = jnp.dot(x_ref[...], w_ref[...], preferred_element_type=jnp.float32)
        y = y * dis[:, None]
        lo_ref[...] = y[:, :DH]
        hi_ref[...] = y[:, DH:]

    return pl.pallas_call(
        body,
        grid=(NBLK,),
        in_specs=[
            pl.BlockSpec((BM, D), lambda i: (i, 0)),
            pl.BlockSpec((D, D), lambda i: (0, 0)),
            pl.BlockSpec((NC, BM), lambda i: (0, i)),
        ],
        out_specs=[
            pl.BlockSpec((BM, DH), lambda i: (i, 0)),
            pl.BlockSpec((BM, DH), lambda i: (i, 0)),
        ],
        out_shape=[
            jax.ShapeDtypeStruct((N_PAD, DH), jnp.float32),
            jax.ShapeDtypeStruct((N_PAD, DH), jnp.float32),
        ],
    )(x, W, deg_parts)


def _bn_stats(accp, xlo, xhi, deg_parts, b):
    def body(a_ref, lo_ref, hi_ref, dp_ref, b_ref, h_ref, s_ref):
        i = pl.program_id(0)
        dis = _dis_from_parts(dp_ref[...])
        agg = jnp.concatenate([a_ref[0] + lo_ref[...],
                               a_ref[1] + hi_ref[...]], axis=-1)
        t = agg * dis[:, None] + b_ref[...]
        h = jnp.maximum(t, 0.0)
        rid = i * BM + lax.broadcasted_iota(jnp.int32, (BM, 1), 0)
        h = jnp.where(rid < N, h, 0.0)
        h_ref[...] = h
        ps = jnp.concatenate(
            [jnp.sum(h, axis=0, keepdims=True),
             jnp.sum(h * h, axis=0, keepdims=True),
             jnp.zeros((6, D), jnp.float32)], axis=0)

        @pl.when(i == 0)
        def _():
            s_ref[...] = ps

        @pl.when(i > 0)
        def _():
            s_ref[...] = s_ref[...] + ps

    return pl.pallas_call(
        body,
        grid=(NBLK,),
        in_specs=[
            pl.BlockSpec((NC, BM, DH), lambda i: (0, i, 0)),
            pl.BlockSpec((BM, DH), lambda i: (i, 0)),
            pl.BlockSpec((BM, DH), lambda i: (i, 0)),
            pl.BlockSpec((NC, BM), lambda i: (0, i)),
            pl.BlockSpec((1, D), lambda i: (0, 0)),
        ],
        out_specs=[
            pl.BlockSpec((BM, D), lambda i: (i, 0)),
            pl.BlockSpec((8, D), lambda i: (0, 0)),
        ],
        out_shape=[
            jax.ShapeDtypeStruct((N_PAD, D), jnp.float32),
            jax.ShapeDtypeStruct((8, D), jnp.float32),
        ],
    )(accp, xlo, xhi, deg_parts, b.reshape(1, D))


def _bn_norm(h_blk, s_blk, g_blk, be_blk, rid):
    mu = s_blk[0] * (1.0 / N)
    var = s_blk[1] * (1.0 / N) - mu * mu
    inv = 1.0 / jnp.sqrt(var + 1e-5)
    xn = g_blk * (h_blk - mu) * inv + be_blk
    return jnp.where(rid < N, xn, 0.0)


def _bn_apply_mat(h, stats, gamma, beta, Wn, deg_parts):
    def body(h_ref, s_ref, g_ref, be_ref, w_ref, dp_ref,
             xn_ref, lo_ref, hi_ref):
        i = pl.program_id(0)
        rid = i * BM + lax.broadcasted_iota(jnp.int32, (BM, 1), 0)
        xn = _bn_norm(h_ref[...], s_ref[...], g_ref[...], be_ref[...], rid)
        xn_ref[...] = xn
        dis = _dis_from_parts(dp_ref[...])
        y = jnp.dot(xn, w_ref[...], preferred_element_type=jnp.float32)
        y = y * dis[:, None]
        lo_ref[...] = y[:, :DH]
        hi_ref[...] = y[:, DH:]

    return pl.pallas_call(
        body,
        grid=(NBLK,),
        in_specs=[
            pl.BlockSpec((BM, D), lambda i: (i, 0)),
            pl.BlockSpec((8, D), lambda i: (0, 0)),
            pl.BlockSpec((1, D), lambda i: (0, 0)),
            pl.BlockSpec((1, D), lambda i: (0, 0)),
            pl.BlockSpec((D, D), lambda i: (0, 0)),
            pl.BlockSpec((NC, BM), lambda i: (0, i)),
        ],
        out_specs=[
            pl.BlockSpec((BM, D), lambda i: (i, 0)),
            pl.BlockSpec((BM, DH), lambda i: (i, 0)),
            pl.BlockSpec((BM, DH), lambda i: (i, 0)),
        ],
        out_shape=[
            jax.ShapeDtypeStruct((N_PAD, D), jnp.float32),
            jax.ShapeDtypeStruct((N_PAD, DH), jnp.float32),
            jax.ShapeDtypeStruct((N_PAD, DH), jnp.float32),
        ],
    )(h, stats, gamma.reshape(1, D), beta.reshape(1, D), Wn, deg_parts)


def _head(h3, st3, gamma3, beta3, x1, x2, lin_W, lin_b):
    def body(h_ref, s_ref, g_ref, be_ref, x1_ref, x2_ref, w_ref, lb_ref, o_ref):
        i = pl.program_id(0)
        rid = i * BM + lax.broadcasted_iota(jnp.int32, (BM, 1), 0)
        x3 = _bn_norm(h_ref[...], s_ref[...], g_ref[...], be_ref[...], rid)
        xc = jnp.concatenate([x1_ref[...], x2_ref[...], x3], axis=-1)
        y = jnp.dot(xc, w_ref[...], preferred_element_type=jnp.float32)
        o_ref[...] = jnp.maximum(y + lb_ref[...], 0.0)

    return pl.pallas_call(
        body,
        grid=(NBLK,),
        in_specs=[
            pl.BlockSpec((BM, D), lambda i: (i, 0)),
            pl.BlockSpec((8, D), lambda i: (0, 0)),
            pl.BlockSpec((1, D), lambda i: (0, 0)),
            pl.BlockSpec((1, D), lambda i: (0, 0)),
            pl.BlockSpec((BM, D), lambda i: (i, 0)),
            pl.BlockSpec((BM, D), lambda i: (i, 0)),
            pl.BlockSpec((3 * D, D), lambda i: (0, 0)),
            pl.BlockSpec((1, D), lambda i: (0, 0)),
        ],
        out_specs=pl.BlockSpec((BM, D), lambda i: (i, 0)),
        out_shape=jax.ShapeDtypeStruct((N_PAD, D), jnp.float32),
    )(h3, st3, gamma3.reshape(1, D), beta3.reshape(1, D), x1, x2,
      lin_W, lin_b.reshape(1, D))


def kernel(x, edge_index, batch, W1, b1, gamma1, beta1, W2, b2, gamma2,
           beta2, W3, b3, gamma3, beta3, lin_W, lin_b):
    x_p = jnp.pad(x, ((0, N_PAD - N), (0, 0)))
    padv = jnp.full((E_PAD - E,), N, jnp.int32)
    src2d = jnp.concatenate([edge_index[0], padv]).reshape(NS * NCHS, CH)
    dst2d = jnp.concatenate([edge_index[1], padv]).reshape(NS * NCHS, CH)
    e2d = jnp.stack([src2d, dst2d], axis=1)
    zrows = jnp.zeros((RPT, DH), jnp.float32)

    deg_parts = _sc_degree(dst2d)

    lo1, hi1 = _mat_scale(x_p, W1, deg_parts)
    acc1 = _sc_scatter(lo1, hi1, e2d, zrows)
    h1, st1 = _bn_stats(acc1, lo1, hi1, deg_parts, b1)

    x1, lo2, hi2 = _bn_apply_mat(h1, st1, gamma1, beta1, W2, deg_parts)
    acc2 = _sc_scatter(lo2, hi2, e2d, zrows)
    h2, st2 = _bn_stats(acc2, lo2, hi2, deg_parts, b2)

    x2, lo3, hi3 = _bn_apply_mat(h2, st2, gamma2, beta2, W3, deg_parts)
    acc3 = _sc_scatter(lo3, hi3, e2d, zrows)
    h3, st3 = _bn_stats(acc3, lo3, hi3, deg_parts, b3)

    out = _head(h3, st3, gamma3, beta3, x1, x2, lin_W, lin_b)
    return out[:N]
